# baseline (speedup 1.0000x reference)
import functools

import jax
import jax.numpy as jnp
from jax import lax
from jax.experimental import pallas as pl
from jax.experimental.pallas import tpu as pltpu
from jax.experimental.pallas import tpu_sc as plsc

D_MODEL = 64
SCALE = 8.0

_NC = 2
_NS = 16
_NW = _NC * _NS

_SB = 256


def _emb_body(xt_hbm, lut_hbm, out_hbm,
              idx0, idx1, rows0, rows1, tb0, tb1,
              si0, si1, sg0, sg1, so0, so1,
              *, units_per_tile, sb_per_t):
  idx = (idx0, idx1)
  rows = (rows0, rows1)
  tbuf = (tb0, tb1)
  sidx = (si0, si1)
  sg = (sg0, sg1)
  so = (so0, so1)

  wid = lax.axis_index("s") * _NC + lax.axis_index("c")
  base = wid * units_per_tile

  def idx_copy(u, s):
    t = u // sb_per_t
    sb = u % sb_per_t
    return pltpu.make_async_copy(
        xt_hbm.at[t, pl.ds(sb * _SB, _SB)], idx[s], sidx[s])

  def gather_copy(s, h):
    return pltpu.make_async_copy(
        lut_hbm.at[idx[s].at[pl.ds(h * 128, 128)]],
        rows[s].at[pl.ds(h * 128, 128)], sg[s])

  def fire_gathers(s):
    for h in range(_SB // 128):
      gather_copy(s, h).start()

  def drain_gathers(s):
    for h in range(_SB // 128):
      gather_copy(s, h).wait()

  def out_copy(u, s, dk):
    t = u // sb_per_t
    sb = u % sb_per_t
    return pltpu.make_async_copy(
        tbuf[s].at[dk], out_hbm.at[t, dk, pl.ds(sb * 2, 2)], so[s])

  def fire_outs(u, s):
    for dk in range(8):
      out_copy(u, s, dk).start()

  def drain_outs(u, s):
    for dk in range(8):
      out_copy(u, s, dk).wait()

  idx_copy(base + 0, 0).start()
  idx_copy(base + 1, 1).start()
  idx_copy(base + 0, 0).wait()
  fire_gathers(0)

  @pl.loop(0, units_per_tile, step=2)
  def _unit_pair(up):
    for b in range(2):
      u = base + up + b
      uu = up + b
      s, t = b, 1 - b

      drain_gathers(s)

      @pl.when(uu + 2 < units_per_tile)
      def _():
        idx_copy(u + 2, s).start()

      @pl.when(uu + 1 < units_per_tile)
      def _():
        @pl.when(uu >= 1)
        def _():
          drain_outs(u - 1, t)
        idx_copy(u + 1, t).wait()
        fire_gathers(t)

      for j in range(_SB // 128):
        @pl.loop(0, 8)
        def _tp(g):
          bidx = j * 128 + g * 16 + lax.iota(jnp.int32, 16)
          for dk in range(8):
            for dr in range(8):
              d = dk * 8 + dr
              v = plsc.load_gather(
                  rows[s], [bidx, jnp.full((16,), d, jnp.int32)])
              tbuf[s][dk, j, dr, pl.ds(g * 16, 16)] = v * SCALE

      fire_outs(u, s)

  drain_outs(base + units_per_tile - 2, 0)
  drain_outs(base + units_per_tile - 1, 1)


@jax.jit
def kernel(x, lut):
  nrow, seq = x.shape
  n_units = seq * (nrow // _SB)
  sb_per_t = nrow // _SB
  assert n_units % _NW == 0
  units_per_tile = n_units // _NW

  xt = jnp.swapaxes(x, 0, 1).astype(jnp.int32)

  mesh = plsc.VectorSubcoreMesh(core_axis_name="c", subcore_axis_name="s")
  body = functools.partial(_emb_body, units_per_tile=units_per_tile,
                           sb_per_t=sb_per_t)
  out5 = pl.kernel(
      body,
      out_type=jax.ShapeDtypeStruct(
          (seq, D_MODEL // 8, nrow // 128, 8, 128), jnp.float32),
      mesh=mesh,
      compiler_params=pltpu.CompilerParams(use_tc_tiling_on_sc=False,
                                           needs_layout_passes=False),
      scratch_types=[
          pltpu.VMEM((_SB,), jnp.int32),
          pltpu.VMEM((_SB,), jnp.int32),
          pltpu.VMEM((_SB, D_MODEL), jnp.float32),
          pltpu.VMEM((_SB, D_MODEL), jnp.float32),
          pltpu.VMEM((8, 2, 8, 128), jnp.float32),
          pltpu.VMEM((8, 2, 8, 128), jnp.float32),
          pltpu.SemaphoreType.DMA,
          pltpu.SemaphoreType.DMA,
          pltpu.SemaphoreType.DMA,
          pltpu.SemaphoreType.DMA,
          pltpu.SemaphoreType.DMA,
          pltpu.SemaphoreType.DMA,
      ],
  )(xt, lut)
  return out5.transpose(2, 4, 0, 1, 3).reshape(nrow, seq, D_MODEL)


# speedup vs baseline: 3.3381x; 3.3381x over previous
import functools

import jax
import jax.numpy as jnp
from jax import lax
from jax.experimental import pallas as pl
from jax.experimental.pallas import tpu as pltpu
from jax.experimental.pallas import tpu_sc as plsc

D_MODEL = 64
SCALE = 8.0

_NC = 2
_NS = 16
_NW = _NC * _NS

_SB = 256


def _emb_body(xt_hbm, lut_hbm, out_hbm,
              idx0, idx1, rows0, rows1, tb0, tb1,
              si0, si1, sg0, sg1, so0, so1,
              *, units_per_tile, sb_per_t):
  idx = (idx0, idx1)
  rows = (rows0, rows1)
  tbuf = (tb0, tb1)
  sidx = (si0, si1)
  sg = (sg0, sg1)
  so = (so0, so1)

  wid = lax.axis_index("s") * _NC + lax.axis_index("c")
  base = wid * units_per_tile

  def idx_copy(u, s):
    t = u // sb_per_t
    sb = u % sb_per_t
    return pltpu.make_async_copy(
        xt_hbm.at[t, pl.ds(sb * _SB, _SB)], idx[s], sidx[s])

  def double_idx(s):
    for q in range(_SB // 16):
      sl = pl.ds(q * 16, 16)
      idx[s][sl] = idx[s][sl] * 2

  def gather_copy(s, h):
    return pltpu.make_async_copy(
        lut_hbm.at[idx[s].at[pl.ds(h * 128, 128)]],
        rows[s].at[pl.ds(h * 128, 128)], sg[s])

  def fire_gathers(s):
    for h in range(_SB // 128):
      gather_copy(s, h).start()

  def drain_gathers(s):
    for h in range(_SB // 128):
      gather_copy(s, h).wait()

  def out_copy(u, s, dk, j):
    t = u // sb_per_t
    sb = u % sb_per_t
    return pltpu.make_async_copy(
        tbuf[s].at[dk * 2 + j], out_hbm.at[t, dk, sb * 2 + j], so[s])

  def fire_outs(u, s):
    for dk in range(8):
      for j in range(2):
        out_copy(u, s, dk, j).start()

  def drain_outs(u, s):
    for dk in range(8):
      for j in range(2):
        out_copy(u, s, dk, j).wait()

  idx_copy(base + 0, 0).start()
  idx_copy(base + 1, 1).start()
  idx_copy(base + 0, 0).wait()
  double_idx(0)
  fire_gathers(0)

  @pl.loop(0, units_per_tile, step=2)
  def _unit_pair(up):
    for b in range(2):
      u = base + up + b
      uu = up + b
      s, t = b, 1 - b

      drain_gathers(s)

      @pl.when(uu + 2 < units_per_tile)
      def _():
        idx_copy(u + 2, s).start()

      @pl.when(uu + 1 < units_per_tile)
      def _():
        @pl.when(uu >= 1)
        def _():
          drain_outs(u - 1, t)
        idx_copy(u + 1, t).wait()
        double_idx(t)
        fire_gathers(t)

      lanes = lax.iota(jnp.int32, 16)
      for k in range(16):
        ck = (lanes + k) & 15
        rh = (ck >> 3) * 2
        drv = ck & 7

        @plsc.parallel_loop(0, _SB // 16)
        def _tp(g):
          bvec = g * 16 + lanes
          blv = (g & 7) * 16 + lanes
          for dq in range(4):
            v = plsc.load_gather(rows[s], [bvec, ck + dq * 16])
            rowv = rh + (dq * 4 + (g >> 3))
            plsc.store_scatter(tbuf[s], [rowv, drv, blv], v * SCALE)

      fire_outs(u, s)

  drain_outs(base + units_per_tile - 2, 0)
  drain_outs(base + units_per_tile - 1, 1)


@jax.jit
def kernel(x, lut):
  nrow, seq = x.shape
  n_units = seq * (nrow // _SB)
  sb_per_t = nrow // _SB
  assert n_units % _NW == 0
  units_per_tile = n_units // _NW

  xt = jnp.swapaxes(x, 0, 1).astype(jnp.int32)
  lut_p = jnp.concatenate(
      [lut, jnp.zeros(lut.shape, lut.dtype)], axis=1).reshape(-1, D_MODEL)

  mesh = plsc.VectorSubcoreMesh(core_axis_name="c", subcore_axis_name="s")
  body = functools.partial(_emb_body, units_per_tile=units_per_tile,
                           sb_per_t=sb_per_t)
  out5 = pl.kernel(
      body,
      out_type=jax.ShapeDtypeStruct(
          (seq, D_MODEL // 8, nrow // 128, 8, 128), jnp.float32),
      mesh=mesh,
      compiler_params=pltpu.CompilerParams(use_tc_tiling_on_sc=False,
                                           needs_layout_passes=False),
      scratch_types=[
          pltpu.VMEM((_SB,), jnp.int32),
          pltpu.VMEM((_SB,), jnp.int32),
          pltpu.VMEM((_SB, D_MODEL), jnp.float32),
          pltpu.VMEM((_SB, D_MODEL), jnp.float32),
          pltpu.VMEM((16, 8, 128), jnp.float32),
          pltpu.VMEM((16, 8, 128), jnp.float32),
          pltpu.SemaphoreType.DMA,
          pltpu.SemaphoreType.DMA,
          pltpu.SemaphoreType.DMA,
          pltpu.SemaphoreType.DMA,
          pltpu.SemaphoreType.DMA,
          pltpu.SemaphoreType.DMA,
      ],
  )(xt, lut_p)
  return out5.transpose(2, 4, 0, 1, 3).reshape(nrow, seq, D_MODEL)
